# Initial kernel scaffold; baseline (speedup 1.0000x reference)
import jax
import jax.numpy as jnp
from jax.experimental import pallas as pl


def kernel(boxes, scores):
    raise NotImplementedError("write your pallas kernel here")


# speedup vs baseline: 3.5937x; 3.5937x over previous
import functools

import jax
import jax.numpy as jnp
from jax import lax
from jax.experimental import pallas as pl
from jax.experimental.pallas import tpu as pltpu
from jax.experimental.pallas import tpu_sc as plsc

N_BOXES = 20000
IOU_THRESHOLD = 0.5
MAX_OUT = 100

NUM_TILES = 16
N_PAD = 20480
PER_TILE = N_PAD // NUM_TILES
CHUNKS = PER_TILE // 16
NEG = float("-inf")
MAXI = 2**31 - 1
NEGBITS = -8388608
OUT_LANES = 16
OUT_FLAT = MAX_OUT * OUT_LANES


def _fused_pass(bx1f, by1f, bx2f, by2f, scv, arv, base, wx1, wy1, wx2, wy2,
                war, first):
    lane = lax.iota(jnp.int32, 16)

    def chunk(j, carry):
        vm, vi = carry
        off = pl.multiple_of(j * 16, 16)
        gsl = pl.ds(base + off, 16)
        sl = pl.ds(off, 16)
        x1 = bx1f[gsl]
        y1 = by1f[gsl]
        x2 = bx2f[gsl]
        y2 = by2f[gsl]
        sc = scv[sl]
        if first:
            ar = (x2 - x1) * (y2 - y1)
            arv[sl] = ar
        else:
            ar = arv[sl]
        xx1 = jnp.maximum(wx1, x1)
        yy1 = jnp.maximum(wy1, y1)
        xx2 = jnp.minimum(wx2, x2)
        yy2 = jnp.minimum(wy2, y2)
        w = jnp.maximum(xx2 - xx1, jnp.float32(0.0))
        h = jnp.maximum(yy2 - yy1, jnp.float32(0.0))
        inter = w * h
        iou = inter / (war + ar - inter + jnp.float32(1e-9))
        ns = jnp.where(iou > jnp.float32(IOU_THRESHOLD), jnp.float32(NEG), sc)
        scv[sl] = ns
        gidx = base + off + lane
        cond = ns > vm
        vm = jnp.where(cond, ns, vm)
        vi = jnp.where(cond, gidx, vi)
        return vm, vi

    vm0 = jnp.full((16,), NEG, jnp.float32)
    vi0 = jnp.full((16,), base, jnp.int32)
    return lax.fori_loop(0, CHUNKS, chunk, (vm0, vi0))


def _nms_body(x1h, y1h, x2h, y2h, sch, outh,
              bx1f, by1f, bx2f, by2f, scv, arv, outv, mbox):
    wid = lax.axis_index("s")
    is_writer = wid == 0
    base = wid * PER_TILE
    lane = lax.iota(jnp.int32, 16)
    zero = jnp.float32(0.0)

    pltpu.sync_copy(x1h, bx1f)
    pltpu.sync_copy(y1h, by1f)
    pltpu.sync_copy(x2h, bx2f)
    pltpu.sync_copy(y2h, by2f)
    pltpu.sync_copy(sch.at[pl.ds(base, PER_TILE)], scv)

    for j in range(4 * NUM_TILES):
        mbox[j] = jnp.int32(0)
    plsc.subcore_barrier()

    vm, vi = _fused_pass(bx1f, by1f, bx2f, by2f, scv, arv, base,
                         zero, zero, zero, zero, zero, first=True)

    def round_body(i, carry):
        vm, vi, ps0, pi0, ps1, pi1 = carry
        vm_bits = plsc.bitcast(vm, jnp.int32)
        lmax_bits = jnp.max(vm_bits)
        lidx = jnp.min(jnp.where(vm_bits == lmax_bits, vi, MAXI))

        bank = jnp.equal(lax.rem(i, 2), 0)
        prev_s = jnp.where(bank, ps0, ps1)
        prev_i = jnp.where(bank, pi0, pi1)
        obank = jnp.where(bank, 0, 2 * NUM_TILES)
        ds_ = lmax_bits - prev_s
        di_ = lidx - prev_i
        slot_s = obank + 2 * wid
        slot_i = slot_s + 1
        def post(t, c):
            plsc.fetch_and_add(mbox.at[slot_s], ds_, subcore_id=t)
            plsc.fetch_and_add(mbox.at[slot_i], di_, subcore_id=t)
            return c

        lax.fori_loop(0, NUM_TILES, post, jnp.int32(0))
        ps0 = jnp.where(bank, lmax_bits, ps0)
        pi0 = jnp.where(bank, lidx, pi0)
        ps1 = jnp.where(bank, ps1, lmax_bits)
        pi1 = jnp.where(bank, pi1, lidx)
        plsc.subcore_barrier()

        bs = mbox[obank]
        bi = mbox[obank + 1]
        for t in range(1, NUM_TILES):
            s_t = mbox[obank + 2 * t]
            i_t = mbox[obank + 2 * t + 1]
            take = jnp.logical_or(
                s_t > bs, jnp.logical_and(s_t == bs, i_t < bi))
            bs = jnp.where(take, s_t, bs)
            bi = jnp.where(take, i_t, bi)
        alive = bs != NEGBITS

        gidxv = jnp.full((16,), bi, jnp.int32)
        wx1 = plsc.load_gather(bx1f, [gidxv])
        wy1 = plsc.load_gather(by1f, [gidxv])
        wx2 = plsc.load_gather(bx2f, [gidxv])
        wy2 = plsc.load_gather(by2f, [gidxv])
        war = (wx2 - wx1) * (wy2 - wy1)
        m_vec = plsc.bitcast(jnp.full((16,), bs, jnp.int32), jnp.float32)

        @pl.when(is_writer)
        def _():
            row = jnp.where(lane == 0, wx1,
                  jnp.where(lane == 1, wy1,
                  jnp.where(lane == 2, wx2,
                  jnp.where(lane == 3, wy2,
                  jnp.where(lane == 4, m_vec, zero)))))
            row = jnp.where(alive, row, zero)
            off = pl.multiple_of(i * OUT_LANES, OUT_LANES)
            outv[pl.ds(off, OUT_LANES)] = row

        vm, vi = _fused_pass(bx1f, by1f, bx2f, by2f, scv, arv, base,
                             wx1, wy1, wx2, wy2, war, first=False)
        return vm, vi, ps0, pi0, ps1, pi1

    zi = jnp.int32(0)
    lax.fori_loop(0, MAX_OUT, round_body, (vm, vi, zi, zi, zi, zi))

    @pl.when(is_writer)
    def _():
        pltpu.sync_copy(outv, outh)


@functools.partial(
    pl.kernel,
    out_type=jax.ShapeDtypeStruct((OUT_FLAT,), jnp.float32),
    mesh=plsc.VectorSubcoreMesh(core_axis_name="c", subcore_axis_name="s",
                                num_cores=1, num_subcores=NUM_TILES),
    compiler_params=pltpu.CompilerParams(needs_layout_passes=False),
    scratch_types=[
        pltpu.VMEM((N_PAD,), jnp.float32),
        pltpu.VMEM((N_PAD,), jnp.float32),
        pltpu.VMEM((N_PAD,), jnp.float32),
        pltpu.VMEM((N_PAD,), jnp.float32),
        pltpu.VMEM((PER_TILE,), jnp.float32),
        pltpu.VMEM((PER_TILE,), jnp.float32),
        pltpu.VMEM((OUT_FLAT,), jnp.float32),
        pltpu.SMEM((4 * NUM_TILES,), jnp.int32),
    ],
)
def _nms_sc(x1h, y1h, x2h, y2h, sch, outh, *scratch):
    _nms_body(x1h, y1h, x2h, y2h, sch, outh, *scratch)


@jax.jit
def kernel(boxes, scores):
    pad = N_PAD - N_BOXES
    x1 = jnp.pad(boxes[:, 0], (0, pad))
    y1 = jnp.pad(boxes[:, 1], (0, pad))
    x2 = jnp.pad(boxes[:, 2], (0, pad))
    y2 = jnp.pad(boxes[:, 3], (0, pad))
    sc = jnp.pad(scores, (0, pad), constant_values=-jnp.inf)
    out_flat = _nms_sc(x1, y1, x2, y2, sc)
    return out_flat.reshape(MAX_OUT, OUT_LANES)[:, :5]


# speedup vs baseline: 10.8313x; 3.0139x over previous
import functools

import jax
import jax.numpy as jnp
from jax import lax
from jax.experimental import pallas as pl
from jax.experimental.pallas import tpu as pltpu
from jax.experimental.pallas import tpu_sc as plsc

N_BOXES = 20000
IOU_THRESHOLD = 0.5
MAX_OUT = 100

NUM_TILES = 16
N_PAD = 20480
PER_TILE = N_PAD // NUM_TILES
CHUNKS = PER_TILE // 16
NEG = float("-inf")
MAXI = 2**31 - 1
NEGBITS = -8388608
OUT_LANES = 16
OUT_FLAT = MAX_OUT * OUT_LANES


def _fused_pass(bx1f, by1f, bx2f, by2f, scv, arv, base, wx1, wy1, wx2, wy2,
                war, first):
    lane = lax.iota(jnp.int32, 16)

    def chunk(j, carry):
        vm, vi = carry
        off = pl.multiple_of(j * 16, 16)
        gsl = pl.ds(base + off, 16)
        sl = pl.ds(off, 16)
        x1 = bx1f[gsl]
        y1 = by1f[gsl]
        x2 = bx2f[gsl]
        y2 = by2f[gsl]
        sc = scv[sl]
        if first:
            ar = (x2 - x1) * (y2 - y1)
            arv[sl] = ar
        else:
            ar = arv[sl]
        xx1 = jnp.maximum(wx1, x1)
        yy1 = jnp.maximum(wy1, y1)
        xx2 = jnp.minimum(wx2, x2)
        yy2 = jnp.minimum(wy2, y2)
        w = jnp.maximum(xx2 - xx1, jnp.float32(0.0))
        h = jnp.maximum(yy2 - yy1, jnp.float32(0.0))
        inter = w * h
        iou = inter / (war + ar - inter + jnp.float32(1e-9))
        ns = jnp.where(iou > jnp.float32(IOU_THRESHOLD), jnp.float32(NEG), sc)
        scv[sl] = ns
        gidx = base + off + lane
        cond = ns > vm
        vm = jnp.where(cond, ns, vm)
        vi = jnp.where(cond, gidx, vi)
        return vm, vi

    vm0 = jnp.full((16,), NEG, jnp.float32)
    vi0 = jnp.full((16,), base, jnp.int32)
    return plsc.parallel_loop(0, CHUNKS, unroll=4, carry=(vm0, vi0))(chunk)


def _nms_body(x1h, y1h, x2h, y2h, sch, outh,
              bx1f, by1f, bx2f, by2f, scv, arv, outv, mbox):
    wid = lax.axis_index("s")
    is_writer = wid == 0
    base = wid * PER_TILE
    lane = lax.iota(jnp.int32, 16)
    zero = jnp.float32(0.0)

    pltpu.sync_copy(x1h, bx1f)
    pltpu.sync_copy(y1h, by1f)
    pltpu.sync_copy(x2h, bx2f)
    pltpu.sync_copy(y2h, by2f)
    pltpu.sync_copy(sch.at[pl.ds(base, PER_TILE)], scv)

    for j in range(4 * NUM_TILES + 4):
        mbox[j] = jnp.int32(0)
    plsc.subcore_barrier()

    vm, vi = _fused_pass(bx1f, by1f, bx2f, by2f, scv, arv, base,
                         zero, zero, zero, zero, zero, first=True)

    def round_body(i, carry):
        vm, vi, ps0, pi0, ps1, pi1 = carry
        vm_bits = plsc.bitcast(vm, jnp.int32)
        lmax_bits = jnp.max(vm_bits)
        lidx = jnp.min(jnp.where(vm_bits == lmax_bits, vi, MAXI))

        bank = jnp.equal(lax.rem(i, 2), 0)
        prev_s = jnp.where(bank, ps0, ps1)
        prev_i = jnp.where(bank, pi0, pi1)
        obank = jnp.where(bank, 0, 2 * NUM_TILES)
        tile0 = jnp.int32(0)
        plsc.fetch_and_add(mbox.at[obank + 2 * wid], lmax_bits - prev_s,
                           subcore_id=tile0)
        plsc.fetch_and_add(mbox.at[obank + 2 * wid + 1], lidx - prev_i,
                           subcore_id=tile0)
        ps0 = jnp.where(bank, lmax_bits, ps0)
        pi0 = jnp.where(bank, lidx, pi0)
        ps1 = jnp.where(bank, ps1, lmax_bits)
        pi1 = jnp.where(bank, pi1, lidx)
        plsc.subcore_barrier()

        rslot = jnp.where(bank, 4 * NUM_TILES, 4 * NUM_TILES + 2)

        @pl.when(is_writer)
        def _():
            cs = mbox[obank]
            ci = mbox[obank + 1]
            for t in range(1, NUM_TILES):
                s_t = mbox[obank + 2 * t]
                i_t = mbox[obank + 2 * t + 1]
                take = jnp.logical_or(
                    s_t > cs, jnp.logical_and(s_t == cs, i_t < ci))
                cs = jnp.where(take, s_t, cs)
                ci = jnp.where(take, i_t, ci)
            mbox[rslot] = cs
            mbox[rslot + 1] = ci

        plsc.subcore_barrier()
        bs = plsc.fetch_and_add(mbox.at[rslot], jnp.int32(0), subcore_id=tile0)
        bi = plsc.fetch_and_add(mbox.at[rslot + 1], jnp.int32(0),
                                subcore_id=tile0)
        alive = bs != NEGBITS

        gidxv = jnp.full((16,), bi, jnp.int32)
        wx1 = plsc.load_gather(bx1f, [gidxv])
        wy1 = plsc.load_gather(by1f, [gidxv])
        wx2 = plsc.load_gather(bx2f, [gidxv])
        wy2 = plsc.load_gather(by2f, [gidxv])
        war = (wx2 - wx1) * (wy2 - wy1)
        m_vec = plsc.bitcast(jnp.full((16,), bs, jnp.int32), jnp.float32)

        @pl.when(is_writer)
        def _():
            row = jnp.where(lane == 0, wx1,
                  jnp.where(lane == 1, wy1,
                  jnp.where(lane == 2, wx2,
                  jnp.where(lane == 3, wy2,
                  jnp.where(lane == 4, m_vec, zero)))))
            row = jnp.where(alive, row, zero)
            off = pl.multiple_of(i * OUT_LANES, OUT_LANES)
            outv[pl.ds(off, OUT_LANES)] = row

        vm, vi = _fused_pass(bx1f, by1f, bx2f, by2f, scv, arv, base,
                             wx1, wy1, wx2, wy2, war, first=False)
        return vm, vi, ps0, pi0, ps1, pi1

    zi = jnp.int32(0)
    lax.fori_loop(0, MAX_OUT, round_body, (vm, vi, zi, zi, zi, zi))

    @pl.when(is_writer)
    def _():
        pltpu.sync_copy(outv, outh)


@functools.partial(
    pl.kernel,
    out_type=jax.ShapeDtypeStruct((OUT_FLAT,), jnp.float32),
    mesh=plsc.VectorSubcoreMesh(core_axis_name="c", subcore_axis_name="s",
                                num_cores=1, num_subcores=NUM_TILES),
    compiler_params=pltpu.CompilerParams(needs_layout_passes=False),
    scratch_types=[
        pltpu.VMEM((N_PAD,), jnp.float32),
        pltpu.VMEM((N_PAD,), jnp.float32),
        pltpu.VMEM((N_PAD,), jnp.float32),
        pltpu.VMEM((N_PAD,), jnp.float32),
        pltpu.VMEM((PER_TILE,), jnp.float32),
        pltpu.VMEM((PER_TILE,), jnp.float32),
        pltpu.VMEM((OUT_FLAT,), jnp.float32),
        pltpu.SMEM((4 * NUM_TILES + 4,), jnp.int32),
    ],
)
def _nms_sc(x1h, y1h, x2h, y2h, sch, outh, *scratch):
    _nms_body(x1h, y1h, x2h, y2h, sch, outh, *scratch)


@jax.jit
def kernel(boxes, scores):
    pad = N_PAD - N_BOXES
    x1 = jnp.pad(boxes[:, 0], (0, pad))
    y1 = jnp.pad(boxes[:, 1], (0, pad))
    x2 = jnp.pad(boxes[:, 2], (0, pad))
    y2 = jnp.pad(boxes[:, 3], (0, pad))
    sc = jnp.pad(scores, (0, pad), constant_values=-jnp.inf)
    out_flat = _nms_sc(x1, y1, x2, y2, sc)
    return out_flat.reshape(MAX_OUT, OUT_LANES)[:, :5]
